# baseline (speedup 1.0000x reference)
import functools

import jax
import jax.numpy as jnp
from jax import lax
from jax.experimental import pallas as pl
from jax.experimental.pallas import tpu as pltpu
from jax.experimental.pallas import tpu_sc as plsc

N_S, N_T, LAT, KNN = 4096, 16384, 128, 16
NG = N_S // 16
TT = 256
TSEL = 1024
TMLP = 256
NW = 32
N_E = N_T * KNN

_DEFAULT = lax.Precision.DEFAULT


def _a_body(lat_ref, pos_ref, wl_ref, wp_ref, bin_ref, a_ref):
    A = lax.dot_general(lat_ref[...], wl_ref[...], (((1,), (0,)), ((), ())),
                        precision=_DEFAULT)
    p = pos_ref[...]
    wp = wp_ref[...]
    P = (p[:, 0:1] * wp[0:1, :] + p[:, 1:2] * wp[1:2, :]
         + p[:, 2:3] * wp[2:3, :])
    a_ref[...] = A - P + bin_ref[...]


def _compute_a(latents, pos, w_lat, w_pos, b_in):
    return pl.pallas_call(
        _a_body,
        out_shape=jax.ShapeDtypeStruct((N_S, LAT), jnp.float32),
    )(latents, pos, w_lat, w_pos, b_in.reshape(1, LAT))


def _knn_body(t_ref, s_ref, st_ref, sp_ref, spt_ref, d2_ref, fidx_ref):
    i = pl.program_id(0)
    t = t_ref[...]
    s = s_ref[...]
    st = st_ref[...]
    M = lax.dot_general(t, s, (((1,), (1,)), ((), ())), precision=_DEFAULT)
    t0, t1, t2 = t[:, 0:1], t[:, 1:2], t[:, 2:3]
    tt = (t0 * t0 + t2 * t2) + t1 * t1
    s0, s1, s2 = st[0:1, :], st[1:2, :], st[2:3, :]
    ss = (s0 * s0 + s2 * s2) + s1 * s1
    d2 = (tt - 2.0 * M) + ss
    d2_ref[...] = d2
    Mp = lax.dot_general(t, sp_ref[...], (((1,), (1,)), ((), ())),
                         precision=_DEFAULT)
    spt = spt_ref[...]
    p0, p1, p2 = spt[0:1, :], spt[1:2, :], spt[2:3, :]
    ssp = (p0 * p0 + p2 * p2) + p1 * p1
    d2p = (tt - 2.0 * Mp) + ssp
    G = jnp.min(d2p.reshape(TT, 16, NG), axis=1)
    giota = lax.broadcasted_iota(jnp.int32, (TT, NG), 1)
    gids = []
    for _ in range(KNN):
        v = jnp.min(G, axis=1, keepdims=True)
        eq = G == v
        gid = jnp.min(jnp.where(eq, giota, NG), axis=1, keepdims=True)
        G = jnp.where(giota == gid, jnp.inf, G)
        gids.append(gid)
    gid16 = jnp.concatenate(gids, axis=1)
    rows = i * TT + lax.broadcasted_iota(jnp.int32, (TT, 1), 0)
    fidx_ref[...] = rows * NG + gid16


def _knn(pnm, pos, pos_t, pos_p, pos_pt):
    return pl.pallas_call(
        _knn_body,
        grid=(N_T // TT,),
        in_specs=[pl.BlockSpec((TT, 3), lambda i: (i, 0)),
                  pl.BlockSpec((N_S, 3), lambda i: (0, 0)),
                  pl.BlockSpec((3, N_S), lambda i: (0, 0)),
                  pl.BlockSpec((N_S, 3), lambda i: (0, 0)),
                  pl.BlockSpec((3, N_S), lambda i: (0, 0))],
        out_specs=[pl.BlockSpec((TT, N_S), lambda i: (i, 0)),
                   pl.BlockSpec((TT, KNN), lambda i: (i, 0))],
        out_shape=[jax.ShapeDtypeStruct((N_T, N_S), jnp.float32),
                   jax.ShapeDtypeStruct((N_T, KNN), jnp.int32)],
    )(pnm, pos, pos_t, pos_p, pos_pt)


_RC = 8
_NBUF = 2


def _gather_c_body(d2v, fidx_hbm, out, fbufs, idxbufs, rowbufs, cbuf, sem):
    wid = lax.axis_index("s") * 2 + lax.axis_index("c")
    rows_per_w = N_T // NW
    base = wid * rows_per_w
    iota16 = lax.iota(jnp.int32, 16)

    def stage(ci, b):
        r0 = base + ci * _RC
        pltpu.sync_copy(fidx_hbm.at[pl.ds(r0, _RC)], fbufs[b])
        for k in range(_RC):
            idxbufs[b][pl.ds(k * 16, 16)] = fbufs[b][k] >> 3
        return pltpu.async_copy(d2v.at[idxbufs[b]], rowbufs[b], sem)

    def process(ci, b, h):
        h.wait()
        for r8 in range(_RC):
            sub = (fbufs[b][r8] & 7) * 16
            rowi = r8 * 16 + iota16
            for j in range(16):
                vals = plsc.load_gather(rowbufs[b], [rowi, sub + j])
                plsc.store_scatter(
                    cbuf, [jnp.full((16,), r8, jnp.int32), iota16 * 16 + j],
                    vals)
        pltpu.sync_copy(cbuf, out.at[pl.ds(base + ci * _RC, _RC)])

    n_chunks = rows_per_w // _RC

    def step(gi, carry):
        handles = [stage(gi * _NBUF + b, b) for b in range(_NBUF)]
        for b in range(_NBUF):
            process(gi * _NBUF + b, b, handles[b])
        return carry

    lax.fori_loop(0, n_chunks // _NBUF, step, 0)


def _gather_c(d2, fidx):
    kern = functools.partial(
        pl.kernel,
        out_type=jax.ShapeDtypeStruct((N_T, NG), jnp.float32),
        mesh=plsc.VectorSubcoreMesh(core_axis_name="c", subcore_axis_name="s"),
        compiler_params=pltpu.CompilerParams(use_tc_tiling_on_sc=True,
                                             needs_layout_passes=False),
        scratch_types=[
            [pltpu.VMEM((_RC, 16), jnp.int32) for _ in range(_NBUF)],
            [pltpu.VMEM((_RC * 16,), jnp.int32) for _ in range(_NBUF)],
            [pltpu.VMEM((_RC * 16, 128), jnp.float32) for _ in range(_NBUF)],
            pltpu.VMEM((_RC, NG), jnp.float32),
            pltpu.SemaphoreType.DMA,
        ],
    )(_gather_c_body)
    return kern(d2.reshape(N_T * 32, 128), fidx)


def _gather_a_body(table, idx2d, out, idxv, rowsv, sem):
    ir_per_w = N_E // 128 // NW
    wid = lax.axis_index("s") * 2 + lax.axis_index("c")
    irbase = wid * ir_per_w
    pltpu.sync_copy(idx2d.at[pl.ds(irbase, ir_per_w)], idxv)
    chunk_ir = 2

    def chunk(ci, carry):
        handles = []
        for j in range(chunk_ir):
            handles.append(pltpu.async_copy(
                table.at[idxv.at[ci * chunk_ir + j]],
                rowsv.at[pl.ds(j * 128, 128)], sem))
        for h in handles:
            h.wait()
        rowstart = (irbase + ci * chunk_ir) * 128
        pltpu.sync_copy(rowsv, out.at[pl.ds(rowstart, chunk_ir * 128)])
        return carry

    lax.fori_loop(0, ir_per_w // chunk_ir, chunk, 0)


def _gather_a(a, cols2d):
    ir_per_w = N_E // 128 // NW
    kern = functools.partial(
        pl.kernel,
        out_type=jax.ShapeDtypeStruct((N_E, LAT), jnp.float32),
        mesh=plsc.VectorSubcoreMesh(core_axis_name="c", subcore_axis_name="s"),
        compiler_params=pltpu.CompilerParams(use_tc_tiling_on_sc=True),
        scratch_types=[pltpu.VMEM((ir_per_w, 128), jnp.int32),
                       pltpu.VMEM((2 * 128, LAT), jnp.float32),
                       pltpu.SemaphoreType.DMA],
    )(_gather_a_body)
    return kern(a, cols2d)


def _sel_body(c_ref, fidx_ref, cols_ref):
    C = c_ref[...]
    gid16 = fidx_ref[...] & (NG - 1)
    siota = lax.broadcasted_iota(jnp.int32, (KNN, NG), 0)
    liota = lax.broadcasted_iota(jnp.int32, (KNN, NG), 1)
    expand = (siota == (liota >> 4)).astype(jnp.float32)
    gexp = lax.dot_general(gid16.astype(jnp.float32), expand,
                           (((1,), (0,)), ((), ())), precision=_DEFAULT)
    jiota = lax.broadcasted_iota(jnp.int32, (TSEL, NG), 1) & 15
    ccols = gexp * 16.0 + jiota.astype(jnp.float32)
    outs = []
    for _ in range(KNN):
        v = jnp.min(C, axis=1, keepdims=True)
        eq = C == v
        col = jnp.min(jnp.where(eq, ccols, float(N_S)), axis=1, keepdims=True)
        C = jnp.where(eq & (ccols == col), jnp.inf, C)
        outs.append(col)
    cols_ref[...] = jnp.concatenate(outs, axis=1).astype(jnp.int32)


def _select(c16, fidx):
    return pl.pallas_call(
        _sel_body,
        grid=(N_T // TSEL,),
        in_specs=[pl.BlockSpec((TSEL, NG), lambda i: (i, 0)),
                  pl.BlockSpec((TSEL, KNN), lambda i: (i, 0))],
        out_specs=pl.BlockSpec((TSEL, KNN), lambda i: (i, 0)),
        out_shape=jax.ShapeDtypeStruct((N_T, KNN), jnp.int32),
    )(c16, fidx)


def _mlp_body(g_ref, t_ref, wp_ref, w1_ref, b1_ref, w2_ref, b2_ref, wo_ref,
              bo_ref, out_ref):
    t = t_ref[...]
    wp = wp_ref[...]
    q = (t[:, 0:1] * wp[0:1, :] + t[:, 1:2] * wp[1:2, :]
         + t[:, 2:3] * wp[2:3, :])
    h = g_ref[...].reshape(TMLP, KNN, LAT) + q[:, None, :]
    x = jnp.maximum(h, 0.0).reshape(TMLP * KNN, LAT)
    x = lax.dot_general(x, w1_ref[...], (((1,), (0,)), ((), ())),
                        precision=_DEFAULT) + b1_ref[...]
    x = jnp.maximum(x, 0.0)
    x = lax.dot_general(x, w2_ref[...], (((1,), (0,)), ((), ())),
                        precision=_DEFAULT) + b2_ref[...]
    y = lax.dot_general(x, wo_ref[...], (((1,), (0,)), ((), ())),
                        precision=_DEFAULT) + bo_ref[...]
    out_ref[...] = y[:, 0]


def _mlp(g, pnm, w_pos, w1, b1, w2, b2, w_out, b_out):
    return pl.pallas_call(
        _mlp_body,
        grid=(N_T // TMLP,),
        in_specs=[pl.BlockSpec((TMLP * KNN, LAT), lambda i: (i, 0)),
                  pl.BlockSpec((TMLP, 3), lambda i: (i, 0)),
                  pl.BlockSpec((3, LAT), lambda i: (0, 0)),
                  pl.BlockSpec((LAT, LAT), lambda i: (0, 0)),
                  pl.BlockSpec((1, LAT), lambda i: (0, 0)),
                  pl.BlockSpec((LAT, LAT), lambda i: (0, 0)),
                  pl.BlockSpec((1, LAT), lambda i: (0, 0)),
                  pl.BlockSpec((LAT, 1), lambda i: (0, 0)),
                  pl.BlockSpec((1, 1), lambda i: (0, 0))],
        out_specs=pl.BlockSpec((TMLP * KNN,), lambda i: (i,)),
        out_shape=jax.ShapeDtypeStruct((N_E,), jnp.float32),
    )(g, pnm, w_pos, w1, b1.reshape(1, LAT), w2, b2.reshape(1, LAT),
      w_out, b_out.reshape(1, 1))


def kernel(pos, batch, pos_non_manifold, pos_non_manifold_batch, latents,
           W_in, b_in, W1, b1, W2, b2, W_out, b_out):
    w_lat = W_in[:LAT]
    w_pos = W_in[LAT:]
    a = _compute_a(latents, pos, w_lat, w_pos, b_in)
    cp = (jnp.arange(N_S, dtype=jnp.int32) % NG) * 16 + (
        jnp.arange(N_S, dtype=jnp.int32) // NG)
    pos_p = pos[cp]
    d2, fidx = _knn(pos_non_manifold, pos, pos.T, pos_p, pos_p.T)
    c = _gather_c(d2, fidx)
    cols = _select(c, fidx)
    g = _gather_a(a, cols.reshape(N_E // 128, 128))
    return _mlp(g, pos_non_manifold, w_pos, W1, b1, W2, b2, W_out, b_out)


# speedup vs baseline: 1.2568x; 1.2568x over previous
import functools

import jax
import jax.numpy as jnp
from jax import lax
from jax.experimental import pallas as pl
from jax.experimental.pallas import tpu as pltpu
from jax.experimental.pallas import tpu_sc as plsc

N_S, N_T, LAT, KNN = 4096, 16384, 128, 16
NG = N_S // 16
TT = 256
TSEL = 1024
TMLP = 256
NW = 32
N_E = N_T * KNN

_DEFAULT = lax.Precision.DEFAULT


def _a_body(lat_ref, pos_ref, wl_ref, wp_ref, bin_ref, a_ref):
    A = lax.dot_general(lat_ref[...], wl_ref[...], (((1,), (0,)), ((), ())),
                        precision=_DEFAULT)
    p = pos_ref[...]
    wp = wp_ref[...]
    P = (p[:, 0:1] * wp[0:1, :] + p[:, 1:2] * wp[1:2, :]
         + p[:, 2:3] * wp[2:3, :])
    a_ref[...] = A - P + bin_ref[...]


def _compute_a(latents, pos, w_lat, w_pos, b_in):
    return pl.pallas_call(
        _a_body,
        out_shape=jax.ShapeDtypeStruct((N_S, LAT), jnp.float32),
    )(latents, pos, w_lat, w_pos, b_in.reshape(1, LAT))


def _knn_body(t_ref, s_ref, st_ref, sp_ref, spt_ref, d2_ref, fidx_ref):
    i = pl.program_id(0)
    t = t_ref[...]
    s = s_ref[...]
    st = st_ref[...]
    M = lax.dot_general(t, s, (((1,), (1,)), ((), ())), precision=_DEFAULT)
    t0, t1, t2 = t[:, 0:1], t[:, 1:2], t[:, 2:3]
    tt = (t0 * t0 + t2 * t2) + t1 * t1
    s0, s1, s2 = st[0:1, :], st[1:2, :], st[2:3, :]
    ss = (s0 * s0 + s2 * s2) + s1 * s1
    d2 = (tt - 2.0 * M) + ss
    d2_ref[...] = d2.reshape(TT, 32, 128)
    Mp = lax.dot_general(t, sp_ref[...], (((1,), (1,)), ((), ())),
                         precision=_DEFAULT)
    spt = spt_ref[...]
    p0, p1, p2 = spt[0:1, :], spt[1:2, :], spt[2:3, :]
    ssp = (p0 * p0 + p2 * p2) + p1 * p1
    d2p = (tt - 2.0 * Mp) + ssp
    G = jnp.min(d2p.reshape(TT, 16, NG), axis=1)
    giota = lax.broadcasted_iota(jnp.int32, (TT, NG), 1)
    gids = []
    for _ in range(KNN):
        v = jnp.min(G, axis=1, keepdims=True)
        eq = G == v
        gid = jnp.min(jnp.where(eq, giota, NG), axis=1, keepdims=True)
        G = jnp.where(giota == gid, jnp.inf, G)
        gids.append(gid)
    gid16 = jnp.concatenate(gids, axis=1)
    rows = i * TT + lax.broadcasted_iota(jnp.int32, (TT, 1), 0)
    fidx_ref[...] = rows * NG + gid16


def _knn(pnm, pos, pos_t, pos_p, pos_pt):
    return pl.pallas_call(
        _knn_body,
        grid=(N_T // TT,),
        in_specs=[pl.BlockSpec((TT, 3), lambda i: (i, 0)),
                  pl.BlockSpec((N_S, 3), lambda i: (0, 0)),
                  pl.BlockSpec((3, N_S), lambda i: (0, 0)),
                  pl.BlockSpec((N_S, 3), lambda i: (0, 0)),
                  pl.BlockSpec((3, N_S), lambda i: (0, 0))],
        out_specs=[pl.BlockSpec((TT, 32, 128), lambda i: (i, 0, 0)),
                   pl.BlockSpec((TT, KNN), lambda i: (i, 0))],
        out_shape=[jax.ShapeDtypeStruct((N_T, 32, 128), jnp.float32),
                   jax.ShapeDtypeStruct((N_T, KNN), jnp.int32)],
    )(pnm, pos, pos_t, pos_p, pos_pt)


_RC = 8
_NBUF = 4


def _gather_c_body(d2v, fidx_hbm, out, fbufs, idxbufs, rowbufs, cbuf, sem):
    wid = lax.axis_index("s") * 2 + lax.axis_index("c")
    rows_per_w = N_T // NW
    base = wid * rows_per_w
    iota16 = lax.iota(jnp.int32, 16)

    def stage(ci, b):
        r0 = base + ci * _RC
        pltpu.sync_copy(fidx_hbm.at[pl.ds(r0, _RC)], fbufs[b])
        for k in range(_RC):
            idxbufs[b][pl.ds(k * 16, 16)] = fbufs[b][k] >> 3
        return pltpu.async_copy(d2v.at[idxbufs[b]], rowbufs[b], sem)

    def process(ci, b, h):
        h.wait()
        for r8 in range(_RC):
            sub = (fbufs[b][r8] & 7) * 16
            rowi = r8 * 16 + iota16
            for j in range(16):
                vals = plsc.load_gather(rowbufs[b], [rowi, sub + j])
                plsc.store_scatter(
                    cbuf, [jnp.full((16,), r8, jnp.int32), iota16 * 16 + j],
                    vals)
        pltpu.sync_copy(cbuf, out.at[pl.ds(base + ci * _RC, _RC)])

    n_chunks = rows_per_w // _RC

    def step(gi, carry):
        handles = [stage(gi * _NBUF + b, b) for b in range(_NBUF)]
        for b in range(_NBUF):
            process(gi * _NBUF + b, b, handles[b])
        return carry

    lax.fori_loop(0, n_chunks // _NBUF, step, 0)


def _gather_c(d2, fidx):
    kern = functools.partial(
        pl.kernel,
        out_type=jax.ShapeDtypeStruct((N_T, NG), jnp.float32),
        mesh=plsc.VectorSubcoreMesh(core_axis_name="c", subcore_axis_name="s"),
        compiler_params=pltpu.CompilerParams(use_tc_tiling_on_sc=True,
                                             needs_layout_passes=False),
        scratch_types=[
            [pltpu.VMEM((_RC, 16), jnp.int32) for _ in range(_NBUF)],
            [pltpu.VMEM((_RC * 16,), jnp.int32) for _ in range(_NBUF)],
            [pltpu.VMEM((_RC * 16, 128), jnp.float32) for _ in range(_NBUF)],
            pltpu.VMEM((_RC, NG), jnp.float32),
            pltpu.SemaphoreType.DMA,
        ],
    )(_gather_c_body)
    return kern(d2.reshape(N_T * 32, 128), fidx)


_ACH = 2
_ANB = 2


def _gather_a_body(table, idx2d, out, idxv, rowbufs, sem):
    ir_per_w = N_E // 128 // NW
    wid = lax.axis_index("s") * 2 + lax.axis_index("c")
    irbase = wid * ir_per_w
    pltpu.sync_copy(idx2d.at[pl.ds(irbase, ir_per_w)], idxv)

    def stage(ci, b):
        return [pltpu.async_copy(table.at[idxv.at[ci * _ACH + j]],
                                 rowbufs[b].at[pl.ds(j * 128, 128)], sem)
                for j in range(_ACH)]

    def process(ci, b, hs):
        for h in hs:
            h.wait()
        rowstart = (irbase + ci * _ACH) * 128
        pltpu.sync_copy(rowbufs[b], out.at[pl.ds(rowstart, _ACH * 128)])

    def step(gi, carry):
        hs = [stage(gi * _ANB + b, b) for b in range(_ANB)]
        for b in range(_ANB):
            process(gi * _ANB + b, b, hs[b])
        return carry

    lax.fori_loop(0, ir_per_w // _ACH // _ANB, step, 0)


def _gather_a(a, cols2d):
    ir_per_w = N_E // 128 // NW
    kern = functools.partial(
        pl.kernel,
        out_type=jax.ShapeDtypeStruct((N_E, LAT), jnp.float32),
        mesh=plsc.VectorSubcoreMesh(core_axis_name="c", subcore_axis_name="s"),
        compiler_params=pltpu.CompilerParams(use_tc_tiling_on_sc=True),
        scratch_types=[pltpu.VMEM((ir_per_w, 128), jnp.int32),
                       [pltpu.VMEM((_ACH * 128, LAT), jnp.float32)
                        for _ in range(_ANB)],
                       pltpu.SemaphoreType.DMA],
    )(_gather_a_body)
    return kern(a, cols2d)


def _sel_body(c_ref, fidx_ref, cols_ref):
    C = c_ref[...]
    gid16 = fidx_ref[...] & (NG - 1)
    siota = lax.broadcasted_iota(jnp.int32, (KNN, NG), 0)
    liota = lax.broadcasted_iota(jnp.int32, (KNN, NG), 1)
    expand = (siota == (liota >> 4)).astype(jnp.float32)
    gexp = lax.dot_general(gid16.astype(jnp.float32), expand,
                           (((1,), (0,)), ((), ())), precision=_DEFAULT)
    jiota = lax.broadcasted_iota(jnp.int32, (TSEL, NG), 1) & 15
    ccols = gexp * 16.0 + jiota.astype(jnp.float32)
    outs = []
    for _ in range(KNN):
        v = jnp.min(C, axis=1, keepdims=True)
        eq = C == v
        col = jnp.min(jnp.where(eq, ccols, float(N_S)), axis=1, keepdims=True)
        C = jnp.where(eq & (ccols == col), jnp.inf, C)
        outs.append(col)
    cols_ref[...] = jnp.concatenate(outs, axis=1).astype(jnp.int32)


def _select(c16, fidx):
    return pl.pallas_call(
        _sel_body,
        grid=(N_T // TSEL,),
        in_specs=[pl.BlockSpec((TSEL, NG), lambda i: (i, 0)),
                  pl.BlockSpec((TSEL, KNN), lambda i: (i, 0))],
        out_specs=pl.BlockSpec((TSEL, KNN), lambda i: (i, 0)),
        out_shape=jax.ShapeDtypeStruct((N_T, KNN), jnp.int32),
    )(c16, fidx)


def _mlp_body(g_ref, t_ref, wp_ref, w1_ref, b1_ref, w2_ref, b2_ref, wo_ref,
              bo_ref, out_ref):
    t = t_ref[...]
    wp = wp_ref[...]
    q = (t[:, 0:1] * wp[0:1, :] + t[:, 1:2] * wp[1:2, :]
         + t[:, 2:3] * wp[2:3, :])
    h = g_ref[...].reshape(TMLP, KNN, LAT) + q[:, None, :]
    x = jnp.maximum(h, 0.0).reshape(TMLP * KNN, LAT)
    x = lax.dot_general(x, w1_ref[...], (((1,), (0,)), ((), ())),
                        precision=_DEFAULT) + b1_ref[...]
    x = jnp.maximum(x, 0.0)
    x = lax.dot_general(x, w2_ref[...], (((1,), (0,)), ((), ())),
                        precision=_DEFAULT) + b2_ref[...]
    y = lax.dot_general(x, wo_ref[...], (((1,), (0,)), ((), ())),
                        precision=_DEFAULT) + bo_ref[...]
    out_ref[...] = y[:, 0]


def _mlp(g, pnm, w_pos, w1, b1, w2, b2, w_out, b_out):
    return pl.pallas_call(
        _mlp_body,
        grid=(N_T // TMLP,),
        in_specs=[pl.BlockSpec((TMLP * KNN, LAT), lambda i: (i, 0)),
                  pl.BlockSpec((TMLP, 3), lambda i: (i, 0)),
                  pl.BlockSpec((3, LAT), lambda i: (0, 0)),
                  pl.BlockSpec((LAT, LAT), lambda i: (0, 0)),
                  pl.BlockSpec((1, LAT), lambda i: (0, 0)),
                  pl.BlockSpec((LAT, LAT), lambda i: (0, 0)),
                  pl.BlockSpec((1, LAT), lambda i: (0, 0)),
                  pl.BlockSpec((LAT, 1), lambda i: (0, 0)),
                  pl.BlockSpec((1, 1), lambda i: (0, 0))],
        out_specs=pl.BlockSpec((TMLP * KNN,), lambda i: (i,)),
        out_shape=jax.ShapeDtypeStruct((N_E,), jnp.float32),
    )(g, pnm, w_pos, w1, b1.reshape(1, LAT), w2, b2.reshape(1, LAT),
      w_out, b_out.reshape(1, 1))


def kernel(pos, batch, pos_non_manifold, pos_non_manifold_batch, latents,
           W_in, b_in, W1, b1, W2, b2, W_out, b_out):
    w_lat = W_in[:LAT]
    w_pos = W_in[LAT:]
    a = _compute_a(latents, pos, w_lat, w_pos, b_in)
    cp = (jnp.arange(N_S, dtype=jnp.int32) % NG) * 16 + (
        jnp.arange(N_S, dtype=jnp.int32) // NG)
    pos_p = pos[cp]
    d2, fidx = _knn(pos_non_manifold, pos, pos.T, pos_p, pos_p.T)
    c = _gather_c(d2, fidx)
    cols = _select(c, fidx)
    g = _gather_a(a, cols.reshape(N_E // 128, 128))
    return _mlp(g, pos_non_manifold, w_pos, W1, b1, W2, b2, W_out, b_out)
